# baseline (speedup 1.0000x reference)
import functools

import jax
import jax.numpy as jnp
from jax import lax
from jax.experimental import pallas as pl
from jax.experimental.pallas import tpu as pltpu
from jax.experimental.pallas import tpu_sc as plsc

_NR = 4


def _gather_mean(x, table):
    B, S = x.shape
    V, E = table.shape
    NC, NS = 2, 16
    NW = NC * NS
    rows_per_w = B // NW
    nvec = E // 16
    splits = ((0, 56), (56, 48), (104, 48), (152, 48))
    mesh = plsc.VectorSubcoreMesh(core_axis_name="c", subcore_axis_name="s")

    @functools.partial(
        pl.kernel,
        mesh=mesh,
        out_type=jax.ShapeDtypeStruct((B, E), jnp.float32),
        compiler_params=pltpu.CompilerParams(use_tc_tiling_on_sc=False),
        scratch_types=[
            pltpu.VMEM((rows_per_w, S), jnp.int32),
            pltpu.VMEM((_NR, S, E), jnp.float32),
            pltpu.VMEM((rows_per_w, E), jnp.float32),
        ]
        + [pltpu.SemaphoreType.DMA] * _NR,
    )
    def k(x_hbm, tab_hbm, out_hbm, idx_v, bufs, out_v, *sems):
        wid = lax.axis_index("s") * NC + lax.axis_index("c")
        rbase = wid * rows_per_w
        pltpu.sync_copy(x_hbm.at[pl.ds(rbase, rows_per_w)], idx_v)

        def fire(r, n):
            for lo, ln in splits:
                pltpu.make_async_copy(
                    tab_hbm.at[idx_v.at[r, pl.ds(lo, ln)]],
                    bufs.at[n, pl.ds(lo, ln)],
                    sems[n],
                ).start()

        def drain(n):
            for lo, ln in splits:
                pltpu.make_async_copy(
                    tab_hbm.at[idx_v.at[0, pl.ds(lo, ln)]],
                    bufs.at[n, pl.ds(lo, ln)],
                    sems[n],
                ).wait()

        def accum(n):
            buf = bufs.at[n]
            unroll = 8
            zero = jnp.zeros((16,), jnp.float32)

            def body(t, a):
                base = t * unroll
                for i in range(unroll):
                    a = tuple(
                        a[q] + buf[base + i, pl.ds(16 * q, 16)]
                        for q in range(nvec)
                    )
                return a

            return lax.fori_loop(0, S // unroll, body, (zero,) * nvec)

        scale = jnp.float32(1.0 / S)

        def do_row(r, n, do_fire):
            drain(n)
            accs = accum(n)
            if do_fire:
                fire(r + _NR, n)
            for q in range(nvec):
                out_v[r, pl.ds(16 * q, 16)] = accs[q] * scale

        for n in range(_NR):
            fire(n, n)

        def loop_body(g, _):
            for n in range(_NR):
                do_row(_NR * g + n, n, True)
            return 0

        lax.fori_loop(0, rows_per_w // _NR - 1, loop_body, 0)
        for n in range(_NR):
            do_row(rows_per_w - _NR + n, n, False)

        pltpu.sync_copy(out_v, out_hbm.at[pl.ds(rbase, rows_per_w)])

    return k(x, table)


def kernel(x, embedding_weight):
    return _gather_mean(x, embedding_weight)


# speedup vs baseline: 1.0012x; 1.0012x over previous
import functools

import jax
import jax.numpy as jnp
from jax import lax
from jax.experimental import pallas as pl
from jax.experimental.pallas import tpu as pltpu
from jax.experimental.pallas import tpu_sc as plsc

_NR = 4


def _gather_mean(x, table):
    B, S = x.shape
    V, E = table.shape
    NC, NS = 2, 16
    NW = NC * NS
    rows_per_w = B // NW
    nvec = E // 16
    splits = ((0, 128), (128, S - 128))
    mesh = plsc.VectorSubcoreMesh(core_axis_name="c", subcore_axis_name="s")

    @functools.partial(
        pl.kernel,
        mesh=mesh,
        out_type=jax.ShapeDtypeStruct((B, E), jnp.float32),
        compiler_params=pltpu.CompilerParams(use_tc_tiling_on_sc=False),
        scratch_types=[
            pltpu.VMEM((rows_per_w, S), jnp.int32),
            pltpu.VMEM((_NR, S, E), jnp.float32),
            pltpu.VMEM((rows_per_w, E), jnp.float32),
        ]
        + [pltpu.SemaphoreType.DMA] * _NR,
    )
    def k(x_hbm, tab_hbm, out_hbm, idx_v, bufs, out_v, *sems):
        wid = lax.axis_index("s") * NC + lax.axis_index("c")
        rbase = wid * rows_per_w
        pltpu.sync_copy(x_hbm.at[pl.ds(rbase, rows_per_w)], idx_v)

        def fire(r, n):
            for lo, ln in splits:
                pltpu.make_async_copy(
                    tab_hbm.at[idx_v.at[r, pl.ds(lo, ln)]],
                    bufs.at[n, pl.ds(lo, ln)],
                    sems[n],
                ).start()

        def drain(n):
            for lo, ln in splits:
                pltpu.make_async_copy(
                    tab_hbm.at[idx_v.at[0, pl.ds(lo, ln)]],
                    bufs.at[n, pl.ds(lo, ln)],
                    sems[n],
                ).wait()

        def accum(n):
            buf = bufs.at[n]
            unroll = 8
            zero = jnp.zeros((16,), jnp.float32)

            def body(t, a):
                base = t * unroll
                for i in range(unroll):
                    a = tuple(
                        a[q] + buf[base + i, pl.ds(16 * q, 16)]
                        for q in range(nvec)
                    )
                return a

            return lax.fori_loop(0, S // unroll, body, (zero,) * nvec)

        scale = jnp.float32(1.0 / S)

        def do_row(r, n, do_fire):
            drain(n)
            accs = accum(n)
            if do_fire:
                fire(r + _NR, n)
            for q in range(nvec):
                out_v[r, pl.ds(16 * q, 16)] = accs[q] * scale

        for n in range(_NR):
            fire(n, n)

        def loop_body(g, _):
            for n in range(_NR):
                do_row(_NR * g + n, n, True)
            return 0

        lax.fori_loop(0, rows_per_w // _NR - 1, loop_body, 0)
        for n in range(_NR):
            do_row(rows_per_w - _NR + n, n, False)

        pltpu.sync_copy(out_v, out_hbm.at[pl.ds(rbase, rows_per_w)])

    return k(x, table)


def kernel(x, embedding_weight):
    return _gather_mean(x, embedding_weight)
